# Initial kernel scaffold; baseline (speedup 1.0000x reference)
import jax
import jax.numpy as jnp
from jax.experimental import pallas as pl


def kernel(user_emb, item_emb, W1, b1, W2, b2, user_inputs, L_inputs, S_inputs, item_inputs):
    raise NotImplementedError("write your pallas kernel here")


# speedup vs baseline: 2.3710x; 2.3710x over previous
import functools

import jax
import jax.numpy as jnp
from jax import lax
from jax.experimental import pallas as pl
from jax.experimental.pallas import tpu as pltpu
from jax.experimental.pallas import tpu_sc as plsc

_B = 1024
_NL = 50
_NS = 20
_D = 128
_H = 16

_CH = 128


def _sc_gather(item_emb, user_emb, idx_item, idx_user):
    info = plsc.get_sparse_core_info()
    nw = info.num_cores * info.num_subcores
    tot = idx_item.shape[0]
    per_w = tot // nw
    n_ch = per_w // _CH
    nu = idx_user.shape[0]
    u_per_w = nu // nw
    mesh = plsc.VectorSubcoreMesh(core_axis_name="c", subcore_axis_name="s")

    @functools.partial(
        pl.kernel,
        mesh=mesh,
        out_type=(
            jax.ShapeDtypeStruct((tot, _D), jnp.float32),
            jax.ShapeDtypeStruct((nu, _D), jnp.float32),
        ),
        scratch_types=[
            pltpu.VMEM((per_w,), jnp.int32),
            pltpu.VMEM((_CH, _D), jnp.float32),
            pltpu.VMEM((_CH, _D), jnp.float32),
            pltpu.VMEM((u_per_w,), jnp.int32),
            pltpu.VMEM((u_per_w, _D), jnp.float32),
            pltpu.SemaphoreType.DMA,
            pltpu.SemaphoreType.DMA,
        ],
    )
    def gather_kernel(item_hbm, user_hbm, idxi_hbm, idxu_hbm, out_hbm, uout_hbm,
                      idx_v, buf0, buf1, uidx_v, urows_v, sem0, sem1):
        wid = lax.axis_index("s") * info.num_cores + lax.axis_index("c")
        base = wid * per_w
        pltpu.sync_copy(idxi_hbm.at[pl.ds(base, per_w)], idx_v)
        bufs = (buf0, buf1)
        sems = (sem0, sem1)
        copies = [None, None]
        for j in range(n_ch):
            k = j % 2
            copies[k] = pltpu.async_copy(
                item_hbm.at[idx_v.at[pl.ds(j * _CH, _CH)]], bufs[k], sems[k])
            if j > 0:
                copies[(j - 1) % 2].wait()
                pltpu.sync_copy(bufs[(j - 1) % 2],
                                out_hbm.at[pl.ds(base + (j - 1) * _CH, _CH)])
        copies[(n_ch - 1) % 2].wait()
        pltpu.sync_copy(bufs[(n_ch - 1) % 2],
                        out_hbm.at[pl.ds(base + (n_ch - 1) * _CH, _CH)])
        ubase = wid * u_per_w
        pltpu.sync_copy(idxu_hbm.at[pl.ds(ubase, u_per_w)], uidx_v)
        pltpu.async_copy(user_hbm.at[uidx_v], urows_v, sem0).wait()
        pltpu.sync_copy(urows_v, uout_hbm.at[pl.ds(ubase, u_per_w)])

    return gather_kernel(item_emb, user_emb, idx_item, idx_user)


def _tc_body(l_ref, s_ref, it_ref, u_ref, w1a_ref, w1b_ref, b1_ref, w2_ref,
             b2_ref, out_ref, *, bb):
    u = u_ref[...]
    a = jnp.dot(u, w1a_ref[...], preferred_element_type=jnp.float32) + b1_ref[...]
    w2v = w2_ref[...]
    b2 = b2_ref[0, 0]

    lf = l_ref[...]
    cl = jnp.dot(lf, w1b_ref[...], preferred_element_type=jnp.float32)
    h = jnp.maximum(cl.reshape(bb, _NL, _H) + a[:, None, :], 0.0)
    sl = jnp.sum(h * w2v[None], axis=-1) + b2
    m = jnp.max(sl, axis=-1, keepdims=True)
    e = jnp.exp(sl - m)
    wl = e / jnp.sum(e, axis=-1, keepdims=True)
    l3 = lf.reshape(bb, _NL, _D)
    u_long = jnp.sum(wl[:, :, None] * l3, axis=1)

    c0 = jnp.dot(u_long, w1b_ref[...], preferred_element_type=jnp.float32)
    h0 = jnp.maximum(a + c0, 0.0)
    s0 = jnp.sum(h0 * w2v, axis=-1, keepdims=True) + b2
    sf = s_ref[...]
    cs = jnp.dot(sf, w1b_ref[...], preferred_element_type=jnp.float32)
    hs = jnp.maximum(cs.reshape(bb, _NS, _H) + a[:, None, :], 0.0)
    ss = jnp.sum(hs * w2v[None], axis=-1) + b2
    m2 = jnp.maximum(jnp.max(ss, axis=-1, keepdims=True), s0)
    e0 = jnp.exp(s0 - m2)
    es = jnp.exp(ss - m2)
    den = e0 + jnp.sum(es, axis=-1, keepdims=True)
    s3 = sf.reshape(bb, _NS, _D)
    hyb = (e0 / den) * u_long + jnp.sum((es / den)[:, :, None] * s3, axis=1)
    out_ref[...] = jnp.sum(hyb * it_ref[...], axis=-1, keepdims=True)


def _tc_compute(gathered, u_rows, w1a_t, w1b_t, b1r, w2r, b2r, *, bb=128,
                interpret=False):
    grid = (_B // bb,)
    l_rows = bb * _NL
    s_rows = bb * _NS
    s_off = (_B * _NL) // s_rows
    i_off = (_B * (_NL + _NS)) // bb
    wspec = lambda shape: pl.BlockSpec(shape, lambda i: (0, 0))
    out2 = pl.pallas_call(
        functools.partial(_tc_body, bb=bb),
        grid=grid,
        in_specs=[
            pl.BlockSpec((l_rows, _D), lambda i: (i, 0)),
            pl.BlockSpec((s_rows, _D), lambda i: (s_off + i, 0)),
            pl.BlockSpec((bb, _D), lambda i: (i_off + i, 0)),
            pl.BlockSpec((bb, _D), lambda i: (i, 0)),
            wspec((_D, _H)),
            wspec((_D, _H)),
            wspec((1, _H)),
            wspec((1, _H)),
            wspec((1, 1)),
        ],
        out_specs=pl.BlockSpec((bb, 1), lambda i: (i, 0)),
        out_shape=jax.ShapeDtypeStruct((_B, 1), jnp.float32),
        interpret=interpret,
    )(gathered, gathered, gathered, u_rows, w1a_t, w1b_t, b1r, w2r, b2r)
    return out2


def kernel(user_emb, item_emb, W1, b1, W2, b2, user_inputs, L_inputs,
           S_inputs, item_inputs):
    idx_item = jnp.concatenate([
        L_inputs.reshape(-1),
        S_inputs.reshape(-1),
        item_inputs,
        jnp.zeros((_B,), item_inputs.dtype),
    ]).astype(jnp.int32)
    idx_user = user_inputs.astype(jnp.int32)

    gathered, u_rows = _sc_gather(item_emb, user_emb, idx_item, idx_user)

    w1a_t = W1[:, :_D].T
    w1b_t = W1[:, _D:].T
    b1r = b1.reshape(1, _H)
    w2r = W2.reshape(1, _H)
    b2r = b2.reshape(1, 1)
    out2 = _tc_compute(gathered, u_rows, w1a_t, w1b_t, b1r, w2r, b2r)
    return out2.reshape(_B, 1, 1)


# speedup vs baseline: 2.5840x; 1.0898x over previous
import functools

import jax
import jax.numpy as jnp
from jax import lax
from jax.experimental import pallas as pl
from jax.experimental.pallas import tpu as pltpu
from jax.experimental.pallas import tpu_sc as plsc

_B = 1024
_NL = 50
_NS = 20
_D = 128
_H = 16

_CH = 128


def _sc_gather(item_emb, user_emb, idx_item, idx_user):
    info = plsc.get_sparse_core_info()
    nw = info.num_cores * info.num_subcores
    tot = idx_item.shape[0]
    per_w = tot // nw
    n_ch = per_w // _CH
    gr = 3
    n_grp = n_ch // gr
    rows_g = gr * _CH
    nu = idx_user.shape[0]
    u_per_w = nu // nw
    mesh = plsc.VectorSubcoreMesh(core_axis_name="c", subcore_axis_name="s")

    @functools.partial(
        pl.kernel,
        mesh=mesh,
        out_type=(
            jax.ShapeDtypeStruct((tot, _D), jnp.float32),
            jax.ShapeDtypeStruct((nu, _D), jnp.float32),
        ),
        scratch_types=[
            pltpu.VMEM((per_w,), jnp.int32),
            pltpu.VMEM((rows_g, _D), jnp.float32),
            pltpu.VMEM((rows_g, _D), jnp.float32),
            pltpu.VMEM((u_per_w,), jnp.int32),
            pltpu.VMEM((u_per_w, _D), jnp.float32),
            pltpu.SemaphoreType.DMA,
            pltpu.SemaphoreType.DMA,
            pltpu.SemaphoreType.DMA,
            pltpu.SemaphoreType.DMA,
            pltpu.SemaphoreType.DMA,
            pltpu.SemaphoreType.DMA,
        ],
    )
    def gather_kernel(item_hbm, user_hbm, idxi_hbm, idxu_hbm, out_hbm, uout_hbm,
                      idx_v, buf0, buf1, uidx_v, urows_v,
                      sg0, sg1, sw0, sw1, su, si):
        wid = lax.axis_index("s") * info.num_cores + lax.axis_index("c")
        base = wid * per_w
        ubase = wid * u_per_w
        ci = pltpu.async_copy(idxi_hbm.at[pl.ds(base, per_w)], idx_v, si)
        cu = pltpu.async_copy(idxu_hbm.at[pl.ds(ubase, u_per_w)], uidx_v, su)
        ci.wait()
        bufs = (buf0, buf1)
        sgs = (sg0, sg1)
        sws = (sw0, sw1)
        gths = [None, None]
        writes = [None, None]
        ug = None
        for g in range(n_grp):
            k = g % 2
            if writes[k] is not None:
                writes[k].wait()
            gths[k] = [
                pltpu.async_copy(
                    item_hbm.at[idx_v.at[pl.ds((g * gr + c) * _CH, _CH)]],
                    bufs[k].at[pl.ds(c * _CH, _CH)], sgs[k])
                for c in range(gr)
            ]
            if g == 0:
                cu.wait()
                ug = pltpu.async_copy(user_hbm.at[uidx_v], urows_v, su)
            if g > 0:
                pk = (g - 1) % 2
                for cp in gths[pk]:
                    cp.wait()
                writes[pk] = pltpu.async_copy(
                    bufs[pk],
                    out_hbm.at[pl.ds(base + (g - 1) * rows_g, rows_g)],
                    sws[pk])
        lk = (n_grp - 1) % 2
        for cp in gths[lk]:
            cp.wait()
        writes[lk] = pltpu.async_copy(
            bufs[lk],
            out_hbm.at[pl.ds(base + (n_grp - 1) * rows_g, rows_g)],
            sws[lk])
        writes[0].wait()
        writes[1].wait()
        ug.wait()
        pltpu.async_copy(urows_v, uout_hbm.at[pl.ds(ubase, u_per_w)], su).wait()

    return gather_kernel(item_emb, user_emb, idx_item, idx_user)


def _tc_body(l_ref, s_ref, it_ref, u_ref, w1a_ref, w1b_ref, b1_ref, w2_ref,
             b2_ref, out_ref, *, bb):
    u = u_ref[...]
    a = jnp.dot(u, w1a_ref[...], preferred_element_type=jnp.float32) + b1_ref[...]
    w2v = w2_ref[...]
    b2 = b2_ref[0, 0]

    lf = l_ref[...]
    cl = jnp.dot(lf, w1b_ref[...], preferred_element_type=jnp.float32)
    h = jnp.maximum(cl.reshape(bb, _NL, _H) + a[:, None, :], 0.0)
    sl = jnp.sum(h * w2v[None], axis=-1) + b2
    m = jnp.max(sl, axis=-1, keepdims=True)
    e = jnp.exp(sl - m)
    wl = e / jnp.sum(e, axis=-1, keepdims=True)
    l3 = lf.reshape(bb, _NL, _D)
    u_long = jnp.sum(wl[:, :, None] * l3, axis=1)

    c0 = jnp.dot(u_long, w1b_ref[...], preferred_element_type=jnp.float32)
    h0 = jnp.maximum(a + c0, 0.0)
    s0 = jnp.sum(h0 * w2v, axis=-1, keepdims=True) + b2
    sf = s_ref[...]
    cs = jnp.dot(sf, w1b_ref[...], preferred_element_type=jnp.float32)
    hs = jnp.maximum(cs.reshape(bb, _NS, _H) + a[:, None, :], 0.0)
    ss = jnp.sum(hs * w2v[None], axis=-1) + b2
    m2 = jnp.maximum(jnp.max(ss, axis=-1, keepdims=True), s0)
    e0 = jnp.exp(s0 - m2)
    es = jnp.exp(ss - m2)
    den = e0 + jnp.sum(es, axis=-1, keepdims=True)
    s3 = sf.reshape(bb, _NS, _D)
    hyb = (e0 / den) * u_long + jnp.sum((es / den)[:, :, None] * s3, axis=1)
    out_ref[...] = jnp.sum(hyb * it_ref[...], axis=-1, keepdims=True)


def _tc_compute(gathered, u_rows, w1a_t, w1b_t, b1r, w2r, b2r, *, bb=128,
                interpret=False):
    grid = (_B // bb,)
    l_rows = bb * _NL
    s_rows = bb * _NS
    s_off = (_B * _NL) // s_rows
    i_off = (_B * (_NL + _NS)) // bb
    wspec = lambda shape: pl.BlockSpec(shape, lambda i: (0, 0))
    out2 = pl.pallas_call(
        functools.partial(_tc_body, bb=bb),
        grid=grid,
        in_specs=[
            pl.BlockSpec((l_rows, _D), lambda i: (i, 0)),
            pl.BlockSpec((s_rows, _D), lambda i: (s_off + i, 0)),
            pl.BlockSpec((bb, _D), lambda i: (i_off + i, 0)),
            pl.BlockSpec((bb, _D), lambda i: (i, 0)),
            wspec((_D, _H)),
            wspec((_D, _H)),
            wspec((1, _H)),
            wspec((1, _H)),
            wspec((1, 1)),
        ],
        out_specs=pl.BlockSpec((bb, 1), lambda i: (i, 0)),
        out_shape=jax.ShapeDtypeStruct((_B, 1), jnp.float32),
        interpret=interpret,
    )(gathered, gathered, gathered, u_rows, w1a_t, w1b_t, b1r, w2r, b2r)
    return out2


def kernel(user_emb, item_emb, W1, b1, W2, b2, user_inputs, L_inputs,
           S_inputs, item_inputs):
    idx_item = jnp.concatenate([
        L_inputs.reshape(-1),
        S_inputs.reshape(-1),
        item_inputs,
        jnp.zeros((_B,), item_inputs.dtype),
    ]).astype(jnp.int32)
    idx_user = user_inputs.astype(jnp.int32)

    gathered, u_rows = _sc_gather(item_emb, user_emb, idx_item, idx_user)

    w1a_t = W1[:, :_D].T
    w1b_t = W1[:, _D:].T
    b1r = b1.reshape(1, _H)
    w2r = W2.reshape(1, _H)
    b2r = b2.reshape(1, 1)
    out2 = _tc_compute(gathered, u_rows, w1a_t, w1b_t, b1r, w2r, b2r)
    return out2.reshape(_B, 1, 1)
